# initial kernel scaffold (unmeasured)
import jax
import jax.numpy as jnp
from jax import lax
from jax.experimental import pallas as pl
from jax.experimental.pallas import tpu as pltpu


def kernel(
    x,
):
    def body(*refs):
        pass

    out_shape = jax.ShapeDtypeStruct(..., jnp.float32)
    return pl.pallas_call(body, out_shape=out_shape)(...)


# baseline (device time: 161446 ns/iter reference)
import jax
import jax.numpy as jnp
from jax import lax
from jax.experimental import pallas as pl
from jax.experimental.pallas import tpu as pltpu

BM = 512


def kernel(x):
    M, N = x.shape
    nblk = M // BM

    def body(x_ref, out_ref, C0, C1, row_halo, col_halo, saved_n,
             sem_in, sem_out, row_send, row_recv, col_send, col_recv):
        sx = lax.axis_index("x")
        sy = lax.axis_index("y")
        C = [C0, C1]

        barrier = pltpu.get_barrier_semaphore()
        pl.semaphore_signal(barrier, inc=1, device_id=(1 - sx, sy),
                            device_id_type=pl.DeviceIdType.MESH)
        pl.semaphore_signal(barrier, inc=1, device_id=(sx, 1 - sy),
                            device_id_type=pl.DeviceIdType.MESH)
        pl.semaphore_wait(barrier, 2)

        row_base = pl.multiple_of(jnp.where(sx == 0, M - 8, 0), 8)
        rdma_row = pltpu.make_async_remote_copy(
            src_ref=x_ref.at[pl.ds(row_base, 8), :], dst_ref=row_halo,
            send_sem=row_send, recv_sem=row_recv,
            device_id=(1 - sx, sy), device_id_type=pl.DeviceIdType.MESH)
        rdma_row.start()
        col_base = pl.multiple_of(jnp.where(sy == 0, N - 128, 0), 128)
        rdma_col = pltpu.make_async_remote_copy(
            src_ref=x_ref.at[:, pl.ds(col_base, 128)], dst_ref=col_halo,
            send_sem=col_send, recv_sem=col_recv,
            device_id=(sx, 1 - sy), device_id_type=pl.DeviceIdType.MESH)
        rdma_col.start()

        cp0 = pltpu.make_async_copy(x_ref.at[pl.ds(0, BM), :], C0, sem_in)
        cp0.start()
        rdma_row.wait()
        rdma_col.wait()
        cp0.wait()

        saved_n[0, :] = row_halo[7, :]

        for i in range(nblk):
            r0 = i * BM
            cur = C[i % 2]
            if i + 1 < nblk:
                cp = pltpu.make_async_copy(
                    x_ref.at[pl.ds(r0 + BM, BM), :], C[(i + 1) % 2], sem_in)
                cp.start()
                cp.wait()
                south = C[(i + 1) % 2][0:1, :]
            else:
                south = row_halo[0:1, :]

            c = cur[:, :]
            n_ = jnp.concatenate([saved_n[0:1, :], c[:BM - 1, :]], axis=0)
            s_ = jnp.concatenate([c[1:, :], south], axis=0)
            hw = col_halo[pl.ds(r0, BM), 127:128]
            he = col_halo[pl.ds(r0, BM), 0:1]
            w_ = jnp.concatenate([hw, c[:, :N - 1]], axis=1)
            e_ = jnp.concatenate([c[:, 1:], he], axis=1)
            res = 0.5 * c + 0.125 * (n_ + s_ + w_ + e_)

            row_io = lax.broadcasted_iota(jnp.int32, (BM, N), 0)
            col_io = lax.broadcasted_iota(jnp.int32, (BM, N), 1)
            mask = ((sy == 0) & (col_io == 0)) | \
                   ((sy == 1) & (col_io == N - 1))
            if i == 0:
                mask = mask | ((sx == 0) & (row_io == 0))
            if i == nblk - 1:
                mask = mask | ((sx == 1) & (row_io == BM - 1))

            saved_n[0, :] = c[BM - 1, :]
            cur[:, :] = jnp.where(mask, c, res)
            cpo = pltpu.make_async_copy(
                cur, out_ref.at[pl.ds(r0, BM), :], sem_out)
            cpo.start()
            cpo.wait()

    return pl.pallas_call(
        body,
        out_shape=jax.ShapeDtypeStruct((M, N), x.dtype),
        in_specs=[pl.BlockSpec(memory_space=pl.ANY)],
        out_specs=pl.BlockSpec(memory_space=pl.ANY),
        scratch_shapes=[
            pltpu.VMEM((BM, N), x.dtype),
            pltpu.VMEM((BM, N), x.dtype),
            pltpu.VMEM((8, N), x.dtype),
            pltpu.VMEM((M, 128), x.dtype),
            pltpu.VMEM((1, N), x.dtype),
            pltpu.SemaphoreType.DMA,
            pltpu.SemaphoreType.DMA,
            pltpu.SemaphoreType.DMA,
            pltpu.SemaphoreType.DMA,
            pltpu.SemaphoreType.DMA,
            pltpu.SemaphoreType.DMA,
        ],
        compiler_params=pltpu.CompilerParams(
            collective_id=0, vmem_limit_bytes=56 * 1024 * 1024),
    )(x)


# device time: 128700 ns/iter; 1.2544x vs baseline; 1.2544x over previous
import jax
import jax.numpy as jnp
from jax import lax
from jax.experimental import pallas as pl
from jax.experimental.pallas import tpu as pltpu

BM = 512
NBUF = 3


def kernel(x):
    M, N = x.shape
    nblk = M // BM

    def body(x_ref, out_ref, C0, C1, C2, strip_col, col_buf,
             row_halo, col_halo, seams, saved_n,
             sems_in, sems_out, sem_strip, sem_seam,
             row_send, row_recv, col_send, col_recv):
        sx = lax.axis_index("x")
        sy = lax.axis_index("y")
        C = [C0, C1, C2]

        barrier = pltpu.get_barrier_semaphore()
        pl.semaphore_signal(barrier, inc=1, device_id=(1 - sx, sy),
                            device_id_type=pl.DeviceIdType.MESH)
        pl.semaphore_signal(barrier, inc=1, device_id=(sx, 1 - sy),
                            device_id_type=pl.DeviceIdType.MESH)
        pl.semaphore_wait(barrier, 2)

        row_base = pl.multiple_of(jnp.where(sx == 0, M - 8, 0), 8)
        rdma_row = pltpu.make_async_remote_copy(
            src_ref=x_ref.at[pl.ds(row_base, 8), :], dst_ref=row_halo,
            send_sem=row_send, recv_sem=row_recv,
            device_id=(1 - sx, sy), device_id_type=pl.DeviceIdType.MESH)
        rdma_row.start()

        col_base = pl.multiple_of(jnp.where(sy == 0, N - 128, 0), 128)
        cp_strip = pltpu.make_async_copy(
            x_ref.at[:, pl.ds(col_base, 128)], strip_col, sem_strip)
        cp_strip.start()

        in_copies = {0: pltpu.make_async_copy(
            x_ref.at[pl.ds(0, BM), :], C0, sems_in.at[0])}
        in_copies[0].start()
        seam_copies = []
        for k in range(1, nblk):
            cp = pltpu.make_async_copy(
                x_ref.at[pl.ds(k * BM, 8), :],
                seams.at[pl.ds(8 * (k - 1), 8), :], sem_seam)
            cp.start()
            seam_copies.append(cp)

        cp_strip.wait()
        col_buf[:, :] = jnp.where(sy == 0, strip_col[:, 127:128],
                                  strip_col[:, 0:1])
        rdma_col = pltpu.make_async_remote_copy(
            src_ref=col_buf, dst_ref=col_halo,
            send_sem=col_send, recv_sem=col_recv,
            device_id=(sx, 1 - sy), device_id_type=pl.DeviceIdType.MESH)
        rdma_col.start()

        for cp in seam_copies:
            cp.wait()
        rdma_row.wait()
        rdma_col.wait()

        saved_n[0, :] = row_halo[7, :]

        out_copies = {}
        for i in range(nblk):
            r0 = i * BM
            cur = C[i % NBUF]
            if i + 1 < nblk:
                if i - 2 >= 0:
                    out_copies[i - 2].wait()
                cp = pltpu.make_async_copy(
                    x_ref.at[pl.ds(r0 + BM, BM), :], C[(i + 1) % NBUF],
                    sems_in.at[(i + 1) % NBUF])
                cp.start()
                in_copies[i + 1] = cp
            in_copies[i].wait()

            north = saved_n[0:1, :]
            if i + 1 < nblk:
                south = seams[8 * i:8 * i + 1, :]
            else:
                south = row_halo[0:1, :]
            c0 = cur[0:1, :]
            c1 = cur[1:2, :]
            cL = cur[BM - 1:BM, :]
            cL1 = cur[BM - 2:BM - 1, :]
            ccol0 = cur[:, 0:1]
            ccol1 = cur[:, 1:2]
            ccolN = cur[:, N - 1:N]
            ccolN1 = cur[:, N - 2:N - 1]
            hcol = col_halo[pl.ds(r0, BM), :]

            rp0 = 0.5 * c0 + 0.125 * (
                north + c1 + pltpu.roll(c0, 1, 1) + pltpu.roll(c0, N - 1, 1))
            rpL = 0.5 * cL + 0.125 * (
                cL1 + south + pltpu.roll(cL, 1, 1) + pltpu.roll(cL, N - 1, 1))
            up0 = jnp.concatenate([north[:, 0:1], ccol0[:BM - 1, :]], axis=0)
            dn0 = jnp.concatenate([ccol0[1:, :], south[:, 0:1]], axis=0)
            colp0 = 0.5 * ccol0 + 0.125 * (up0 + dn0 + hcol + ccol1)
            upN = jnp.concatenate([north[:, N - 1:N], ccolN[:BM - 1, :]],
                                  axis=0)
            dnN = jnp.concatenate([ccolN[1:, :], south[:, N - 1:N]], axis=0)
            colpN = 0.5 * ccolN + 0.125 * (upN + dnN + ccolN1 + hcol)

            c = cur[:, :]
            t = pltpu.roll(c, 1, 0)
            t = t + pltpu.roll(c, BM - 1, 0)
            t = t + pltpu.roll(c, 1, 1)
            t = t + pltpu.roll(c, N - 1, 1)
            saved_n[0, :] = cL[0, :]
            cur[:, :] = 0.5 * c + 0.125 * t

            cur[0:1, :] = rp0
            cur[BM - 1:BM, :] = rpL
            cur[:, 0:1] = colp0
            cur[:, N - 1:N] = colpN

            @pl.when(sy == 0)
            def _():
                cur[:, 0:1] = ccol0

            @pl.when(sy == 1)
            def _():
                cur[:, N - 1:N] = ccolN

            if i == 0:
                @pl.when(sx == 0)
                def _():
                    cur[0:1, :] = c0

            if i == nblk - 1:
                @pl.when(sx == 1)
                def _():
                    cur[BM - 1:BM, :] = cL

            cpo = pltpu.make_async_copy(
                cur, out_ref.at[pl.ds(r0, BM), :], sems_out.at[i % NBUF])
            cpo.start()
            out_copies[i] = cpo

        for i in range(nblk - NBUF, nblk):
            out_copies[i].wait()

    return pl.pallas_call(
        body,
        out_shape=jax.ShapeDtypeStruct((M, N), x.dtype),
        in_specs=[pl.BlockSpec(memory_space=pl.ANY)],
        out_specs=pl.BlockSpec(memory_space=pl.ANY),
        scratch_shapes=[
            pltpu.VMEM((BM, N), x.dtype),
            pltpu.VMEM((BM, N), x.dtype),
            pltpu.VMEM((BM, N), x.dtype),
            pltpu.VMEM((M, 128), x.dtype),
            pltpu.VMEM((M, 1), x.dtype),
            pltpu.VMEM((8, N), x.dtype),
            pltpu.VMEM((M, 1), x.dtype),
            pltpu.VMEM((8 * (M // BM - 1), N), x.dtype),
            pltpu.VMEM((1, N), x.dtype),
            pltpu.SemaphoreType.DMA((NBUF,)),
            pltpu.SemaphoreType.DMA((NBUF,)),
            pltpu.SemaphoreType.DMA,
            pltpu.SemaphoreType.DMA,
            pltpu.SemaphoreType.DMA,
            pltpu.SemaphoreType.DMA,
            pltpu.SemaphoreType.DMA,
            pltpu.SemaphoreType.DMA,
        ],
        compiler_params=pltpu.CompilerParams(
            collective_id=0, vmem_limit_bytes=60 * 1024 * 1024),
    )(x)


# device time: 122736 ns/iter; 1.3154x vs baseline; 1.0486x over previous
import jax
import jax.numpy as jnp
from jax import lax
from jax.experimental import pallas as pl
from jax.experimental.pallas import tpu as pltpu

BM = 512
NBUF = 3


def kernel(x):
    M, N = x.shape
    nblk = M // BM

    def body(x_ref, out_ref, C0, C1, C2, strip_col, col_buf,
             row_halo, col_halo, seams, saved_n,
             sems_in, sems_out, sem_strip, sem_seam,
             row_send, row_recv, col_send, col_recv):
        sx = lax.axis_index("x")
        sy = lax.axis_index("y")
        C = [C0, C1, C2]

        barrier = pltpu.get_barrier_semaphore()
        pl.semaphore_signal(barrier, inc=1, device_id=(1 - sx, sy),
                            device_id_type=pl.DeviceIdType.MESH)
        pl.semaphore_signal(barrier, inc=1, device_id=(sx, 1 - sy),
                            device_id_type=pl.DeviceIdType.MESH)
        pl.semaphore_wait(barrier, 2)

        row_base = pl.multiple_of(jnp.where(sx == 0, M - 8, 0), 8)
        rdma_row = pltpu.make_async_remote_copy(
            src_ref=x_ref.at[pl.ds(row_base, 8), :], dst_ref=row_halo,
            send_sem=row_send, recv_sem=row_recv,
            device_id=(1 - sx, sy), device_id_type=pl.DeviceIdType.MESH)
        rdma_row.start()

        col_base = pl.multiple_of(jnp.where(sy == 0, N - 128, 0), 128)
        cp_strip = pltpu.make_async_copy(
            x_ref.at[:, pl.ds(col_base, 128)], strip_col, sem_strip)
        cp_strip.start()

        in_copies = {0: pltpu.make_async_copy(
            x_ref.at[pl.ds(0, BM), :], C0, sems_in.at[0])}
        in_copies[0].start()
        seam_copies = []
        for k in range(1, nblk):
            cp = pltpu.make_async_copy(
                x_ref.at[pl.ds(k * BM, 8), :],
                seams.at[pl.ds(8 * (k - 1), 8), :], sem_seam)
            cp.start()
            seam_copies.append(cp)

        cp_strip.wait()
        col_buf[:, :] = jnp.where(sy == 0, strip_col[:, 127:128],
                                  strip_col[:, 0:1])
        rdma_col = pltpu.make_async_remote_copy(
            src_ref=col_buf, dst_ref=col_halo,
            send_sem=col_send, recv_sem=col_recv,
            device_id=(sx, 1 - sy), device_id_type=pl.DeviceIdType.MESH)
        rdma_col.start()

        for cp in seam_copies:
            cp.wait()
        rdma_row.wait()
        rdma_col.wait()

        saved_n[0, :] = row_halo[7, :]

        ii = lax.broadcasted_iota(jnp.int32, (BM, BM), 0)
        jj = lax.broadcasted_iota(jnp.int32, (BM, BM), 1)
        d = ii - jj
        Trow = jnp.where(d == 0, jnp.float32(0.5), jnp.float32(0.0)) + \
            jnp.where(jnp.abs(d) == 1, jnp.float32(0.125), jnp.float32(0.0))
        TT = BM
        dt = lax.broadcasted_iota(jnp.int32, (TT, TT), 0) - \
            lax.broadcasted_iota(jnp.int32, (TT, TT), 1)
        T2 = jnp.where(jnp.abs(dt) == 1, jnp.float32(0.125),
                       jnp.float32(0.0))
        ntile = N // TT

        out_copies = {}
        for i in range(nblk):
            r0 = i * BM
            cur = C[i % NBUF]
            if i + 1 < nblk:
                if i - 2 >= 0:
                    out_copies[i - 2].wait()
                cp = pltpu.make_async_copy(
                    x_ref.at[pl.ds(r0 + BM, BM), :], C[(i + 1) % NBUF],
                    sems_in.at[(i + 1) % NBUF])
                cp.start()
                in_copies[i + 1] = cp
            in_copies[i].wait()

            north = saved_n[0:1, :]
            if i + 1 < nblk:
                south = seams[8 * i:8 * i + 1, :]
            else:
                south = row_halo[0:1, :]
            c0 = cur[0:1, :]
            c1 = cur[1:2, :]
            cL = cur[BM - 1:BM, :]
            cL1 = cur[BM - 2:BM - 1, :]
            ccol0 = cur[:, 0:1]
            ccol1 = cur[:, 1:2]
            ccolN = cur[:, N - 1:N]
            ccolN1 = cur[:, N - 2:N - 1]
            hcol = col_halo[pl.ds(r0, BM), :]
            seamL = {t: cur[:, t * TT - 1:t * TT] for t in range(1, ntile)}
            seamR = {t: cur[:, t * TT:t * TT + 1] for t in range(1, ntile)}

            rp0 = 0.5 * c0 + 0.125 * (
                north + c1 + pltpu.roll(c0, 1, 1) + pltpu.roll(c0, N - 1, 1))
            rpL = 0.5 * cL + 0.125 * (
                cL1 + south + pltpu.roll(cL, 1, 1) + pltpu.roll(cL, N - 1, 1))
            up0 = jnp.concatenate([north[:, 0:1], ccol0[:BM - 1, :]], axis=0)
            dn0 = jnp.concatenate([ccol0[1:, :], south[:, 0:1]], axis=0)
            colp0 = 0.5 * ccol0 + 0.125 * (up0 + dn0 + hcol + ccol1)
            upN = jnp.concatenate([north[:, N - 1:N], ccolN[:BM - 1, :]],
                                  axis=0)
            dnN = jnp.concatenate([ccolN[1:, :], south[:, N - 1:N]], axis=0)
            colpN = 0.5 * ccolN + 0.125 * (upN + dnN + ccolN1 + hcol)

            saved_n[0, :] = cL[0, :]

            for t in range(ntile):
                sl = slice(t * TT, (t + 1) * TT)
                ct = cur[:, sl]
                cur[:, sl] = (
                    jnp.dot(Trow, ct, preferred_element_type=jnp.float32)
                    + jnp.dot(ct, T2, preferred_element_type=jnp.float32))
            for t in range(1, ntile):
                cur[:, t * TT:t * TT + 1] = \
                    cur[:, t * TT:t * TT + 1] + 0.125 * seamL[t]
                cur[:, t * TT - 1:t * TT] = \
                    cur[:, t * TT - 1:t * TT] + 0.125 * seamR[t]

            cur[0:1, :] = rp0
            cur[BM - 1:BM, :] = rpL
            cur[:, 0:1] = colp0
            cur[:, N - 1:N] = colpN

            @pl.when(sy == 0)
            def _():
                cur[:, 0:1] = ccol0

            @pl.when(sy == 1)
            def _():
                cur[:, N - 1:N] = ccolN

            if i == 0:
                @pl.when(sx == 0)
                def _():
                    cur[0:1, :] = c0

            if i == nblk - 1:
                @pl.when(sx == 1)
                def _():
                    cur[BM - 1:BM, :] = cL

            cpo = pltpu.make_async_copy(
                cur, out_ref.at[pl.ds(r0, BM), :], sems_out.at[i % NBUF])
            cpo.start()
            out_copies[i] = cpo

        for i in range(nblk - NBUF, nblk):
            out_copies[i].wait()

    return pl.pallas_call(
        body,
        out_shape=jax.ShapeDtypeStruct((M, N), x.dtype),
        in_specs=[pl.BlockSpec(memory_space=pl.ANY)],
        out_specs=pl.BlockSpec(memory_space=pl.ANY),
        scratch_shapes=[
            pltpu.VMEM((BM, N), x.dtype),
            pltpu.VMEM((BM, N), x.dtype),
            pltpu.VMEM((BM, N), x.dtype),
            pltpu.VMEM((M, 128), x.dtype),
            pltpu.VMEM((M, 1), x.dtype),
            pltpu.VMEM((8, N), x.dtype),
            pltpu.VMEM((M, 1), x.dtype),
            pltpu.VMEM((8 * (M // BM - 1), N), x.dtype),
            pltpu.VMEM((1, N), x.dtype),
            pltpu.SemaphoreType.DMA((NBUF,)),
            pltpu.SemaphoreType.DMA((NBUF,)),
            pltpu.SemaphoreType.DMA,
            pltpu.SemaphoreType.DMA,
            pltpu.SemaphoreType.DMA,
            pltpu.SemaphoreType.DMA,
            pltpu.SemaphoreType.DMA,
            pltpu.SemaphoreType.DMA,
        ],
        compiler_params=pltpu.CompilerParams(
            collective_id=0, vmem_limit_bytes=60 * 1024 * 1024),
    )(x)


# device time: 116227 ns/iter; 1.3891x vs baseline; 1.0560x over previous
import jax
import jax.numpy as jnp
from jax import lax
from jax.experimental import pallas as pl
from jax.experimental.pallas import tpu as pltpu

BM = 512


def kernel(x):
    M, N = x.shape
    nblk = M // BM
    spb = BM // 8

    def body(x_ref, nseam_ref, sseam_ref, top_ref, btm_ref,
             o_ref, row_buf, col_piece, row_halo, col_halo,
             row_send, row_recv, col_send, col_recvs):
        i = pl.program_id(0)
        sx = lax.axis_index("x")
        sy = lax.axis_index("y")

        @pl.when(i == 0)
        def _():
            barrier = pltpu.get_barrier_semaphore()
            pl.semaphore_signal(barrier, inc=1, device_id=(1 - sx, sy),
                                device_id_type=pl.DeviceIdType.MESH)
            pl.semaphore_signal(barrier, inc=1, device_id=(sx, 1 - sy),
                                device_id_type=pl.DeviceIdType.MESH)
            pl.semaphore_wait(barrier, 2)

            row_buf[0, :] = jnp.where(sx == 0, btm_ref[7, :], top_ref[0, :])
            rdma_row = pltpu.make_async_remote_copy(
                src_ref=row_buf, dst_ref=row_halo,
                send_sem=row_send, recv_sem=row_recv,
                device_id=(1 - sx, sy),
                device_id_type=pl.DeviceIdType.MESH)
            rdma_row.start()
            rdma_row.wait()

        c = x_ref[:, :]

        col_piece[:, :] = jnp.where(sy == 0, c[:, N - 1:N], c[:, 0:1])
        rdma_col = pltpu.make_async_remote_copy(
            src_ref=col_piece,
            dst_ref=col_halo.at[pl.ds(pl.multiple_of(i * BM, 8), BM), :],
            send_sem=col_send, recv_sem=col_recvs.at[i],
            device_id=(sx, 1 - sy), device_id_type=pl.DeviceIdType.MESH)
        rdma_col.start()

        north = jnp.where(i == 0, row_halo[0:1, :], nseam_ref[7:8, :])
        south = jnp.where(i == nblk - 1, row_halo[0:1, :], sseam_ref[0:1, :])
        c0 = c[0:1, :]
        cL = c[BM - 1:BM, :]
        ccol0 = c[:, 0:1]
        ccolN = c[:, N - 1:N]

        o_ref[:, :] = 0.5 * c + 0.125 * (
            pltpu.roll(c, 1, 0) + pltpu.roll(c, BM - 1, 0)
            + pltpu.roll(c, 1, 1) + pltpu.roll(c, N - 1, 1))

        o_ref[0:1, :] = o_ref[0:1, :] + 0.125 * (north - cL)
        o_ref[BM - 1:BM, :] = o_ref[BM - 1:BM, :] + 0.125 * (south - c0)

        rdma_col.wait_recv()
        hcol = col_halo[0:BM, :]
        for k in range(1, nblk):
            hcol = jnp.where(i == k, col_halo[k * BM:(k + 1) * BM, :], hcol)
        o_ref[:, 0:1] = o_ref[:, 0:1] + 0.125 * (hcol - ccolN)
        o_ref[:, N - 1:N] = o_ref[:, N - 1:N] + 0.125 * (hcol - ccol0)

        @pl.when(sy == 0)
        def _():
            o_ref[:, 0:1] = ccol0

        @pl.when(sy == 1)
        def _():
            o_ref[:, N - 1:N] = ccolN

        @pl.when((sx == 0) & (i == 0))
        def _():
            o_ref[0:1, :] = c0

        @pl.when((sx == 1) & (i == nblk - 1))
        def _():
            o_ref[BM - 1:BM, :] = cL

        rdma_col.wait_send()

    return pl.pallas_call(
        body,
        grid=(nblk,),
        in_specs=[
            pl.BlockSpec((BM, N), lambda i: (i, 0)),
            pl.BlockSpec((8, N),
                         lambda i: (jnp.maximum(i * spb - 1, 0), 0)),
            pl.BlockSpec((8, N),
                         lambda i: (jnp.minimum((i + 1) * spb,
                                                M // 8 - 1), 0)),
            pl.BlockSpec((8, N), lambda i: (0, 0)),
            pl.BlockSpec((8, N), lambda i: (M // 8 - 1, 0)),
        ],
        out_specs=pl.BlockSpec((BM, N), lambda i: (i, 0)),
        out_shape=jax.ShapeDtypeStruct((M, N), x.dtype),
        scratch_shapes=[
            pltpu.VMEM((1, N), jnp.float32),
            pltpu.VMEM((BM, 1), jnp.float32),
            pltpu.VMEM((1, N), jnp.float32),
            pltpu.VMEM((M, 1), jnp.float32),
            pltpu.SemaphoreType.DMA,
            pltpu.SemaphoreType.DMA,
            pltpu.SemaphoreType.DMA,
            pltpu.SemaphoreType.DMA((8,)),
        ],
        compiler_params=pltpu.CompilerParams(
            collective_id=0, vmem_limit_bytes=58 * 1024 * 1024),
    )(x, x, x, x, x)
